# baseline (speedup 1.0000x reference)
import jax
import jax.numpy as jnp
from jax.experimental import pallas as pl
from jax.experimental.pallas import tpu as pltpu

N = 10000
K = 32
D = 128
BN = 640


def _body(self_ref, nf_ref, ws_ref, bs_ref, wn_ref, bn_ref, g_ref,
          beta_ref, out_ref):
    agg = jnp.sum(nf_ref[...], axis=0) * (1.0 / K)
    out = (
        jax.lax.dot_general(self_ref[...], ws_ref[...],
                            (((1,), (1,)), ((), ())),
                            preferred_element_type=jnp.float32)
        + jax.lax.dot_general(agg, wn_ref[...], (((1,), (1,)), ((), ())),
                              preferred_element_type=jnp.float32)
        + bs_ref[...] + bn_ref[...]
    )
    mu = jnp.mean(out, axis=-1, keepdims=True)
    var = jnp.mean(jnp.square(out - mu), axis=-1, keepdims=True)
    normed = (out - mu) * jax.lax.rsqrt(var + 1e-5) * g_ref[...] + beta_ref[...]
    out_ref[...] = jnp.maximum(normed, 0.0)


@jax.jit
def kernel(self_feat, neighbor_feats, W_self, b_self, W_nb, b_nb, ln_gamma, ln_beta):
    vec = pl.BlockSpec((D,), lambda i: (0,))
    mat = pl.BlockSpec((D, D), lambda i: (0, 0))
    return pl.pallas_call(
        _body,
        grid=(pl.cdiv(N, BN),),
        in_specs=[
            pl.BlockSpec((BN, D), lambda i: (i, 0)),
            pl.BlockSpec((K, BN, D), lambda i: (0, i, 0)),
            mat, vec, mat, vec, vec, vec,
        ],
        out_specs=pl.BlockSpec((BN, D), lambda i: (i, 0)),
        out_shape=jax.ShapeDtypeStruct((N, D), jnp.float32),
        compiler_params=pltpu.CompilerParams(
            dimension_semantics=("arbitrary",),
        ),
    )(self_feat, neighbor_feats, W_self, b_self, W_nb, b_nb, ln_gamma,
      ln_beta)


# speedup vs baseline: 1.0063x; 1.0063x over previous
import jax
import jax.numpy as jnp
from jax.experimental import pallas as pl
from jax.experimental.pallas import tpu as pltpu

N = 10000
K = 32
D = 128
BN = 560


def _body(self_ref, nf_ref, ws_ref, bs_ref, wn_ref, bn_ref, g_ref,
          beta_ref, out_ref):
    agg = jnp.sum(nf_ref[...], axis=0) * (1.0 / K)
    out = (
        jax.lax.dot_general(self_ref[...], ws_ref[...],
                            (((1,), (1,)), ((), ())),
                            preferred_element_type=jnp.float32)
        + jax.lax.dot_general(agg, wn_ref[...], (((1,), (1,)), ((), ())),
                              preferred_element_type=jnp.float32)
        + bs_ref[...] + bn_ref[...]
    )
    mu = jnp.mean(out, axis=-1, keepdims=True)
    var = jnp.mean(jnp.square(out - mu), axis=-1, keepdims=True)
    normed = (out - mu) * jax.lax.rsqrt(var + 1e-5) * g_ref[...] + beta_ref[...]
    out_ref[...] = jnp.maximum(normed, 0.0)


@jax.jit
def kernel(self_feat, neighbor_feats, W_self, b_self, W_nb, b_nb, ln_gamma, ln_beta):
    vec = pl.BlockSpec((D,), lambda i: (0,))
    mat = pl.BlockSpec((D, D), lambda i: (0, 0))
    return pl.pallas_call(
        _body,
        grid=(pl.cdiv(N, BN),),
        in_specs=[
            pl.BlockSpec((BN, D), lambda i: (i, 0)),
            pl.BlockSpec((K, BN, D), lambda i: (0, i, 0)),
            mat, vec, mat, vec, vec, vec,
        ],
        out_specs=pl.BlockSpec((BN, D), lambda i: (i, 0)),
        out_shape=jax.ShapeDtypeStruct((N, D), jnp.float32),
        compiler_params=pltpu.CompilerParams(
            dimension_semantics=("arbitrary",),
        ),
    )(self_feat, neighbor_feats, W_self, b_self, W_nb, b_nb, ln_gamma,
      ln_beta)


# speedup vs baseline: 1.0095x; 1.0032x over previous
import jax
import jax.numpy as jnp
from jax.experimental import pallas as pl
from jax.experimental.pallas import tpu as pltpu

N = 10000
K = 32
D = 128
BN = 480


def _body(self_ref, nf_ref, ws_ref, bs_ref, wn_ref, bn_ref, g_ref,
          beta_ref, out_ref):
    agg = jnp.sum(nf_ref[...], axis=0) * (1.0 / K)
    out = (
        jax.lax.dot_general(self_ref[...], ws_ref[...],
                            (((1,), (1,)), ((), ())),
                            preferred_element_type=jnp.float32)
        + jax.lax.dot_general(agg, wn_ref[...], (((1,), (1,)), ((), ())),
                              preferred_element_type=jnp.float32)
        + bs_ref[...] + bn_ref[...]
    )
    mu = jnp.mean(out, axis=-1, keepdims=True)
    var = jnp.mean(jnp.square(out - mu), axis=-1, keepdims=True)
    normed = (out - mu) * jax.lax.rsqrt(var + 1e-5) * g_ref[...] + beta_ref[...]
    out_ref[...] = jnp.maximum(normed, 0.0)


@jax.jit
def kernel(self_feat, neighbor_feats, W_self, b_self, W_nb, b_nb, ln_gamma, ln_beta):
    vec = pl.BlockSpec((D,), lambda i: (0,))
    mat = pl.BlockSpec((D, D), lambda i: (0, 0))
    return pl.pallas_call(
        _body,
        grid=(pl.cdiv(N, BN),),
        in_specs=[
            pl.BlockSpec((BN, D), lambda i: (i, 0)),
            pl.BlockSpec((K, BN, D), lambda i: (0, i, 0)),
            mat, vec, mat, vec, vec, vec,
        ],
        out_specs=pl.BlockSpec((BN, D), lambda i: (i, 0)),
        out_shape=jax.ShapeDtypeStruct((N, D), jnp.float32),
        compiler_params=pltpu.CompilerParams(
            dimension_semantics=("parallel",),
        ),
    )(self_feat, neighbor_feats, W_self, b_self, W_nb, b_nb, ln_gamma,
      ln_beta)


# speedup vs baseline: 1.0095x; 1.0000x over previous
import jax
import jax.numpy as jnp
from jax.experimental import pallas as pl
from jax.experimental.pallas import tpu as pltpu

N = 10000
K = 32
D = 128
BN = 480


def _body(self_ref, nf_ref, ws_ref, bs_ref, wn_ref, bn_ref, g_ref,
          beta_ref, out_ref):
    agg = jnp.sum(nf_ref[...], axis=0) * (1.0 / K)
    out = (
        jax.lax.dot_general(self_ref[...], ws_ref[...],
                            (((1,), (1,)), ((), ())),
                            preferred_element_type=jnp.float32)
        + jax.lax.dot_general(agg, wn_ref[...], (((1,), (1,)), ((), ())),
                              preferred_element_type=jnp.float32)
        + bs_ref[...] + bn_ref[...]
    )
    mu = jnp.mean(out, axis=-1, keepdims=True)
    var = jnp.mean(jnp.square(out - mu), axis=-1, keepdims=True)
    normed = (out - mu) * jax.lax.rsqrt(var + 1e-5) * g_ref[...] + beta_ref[...]
    out_ref[...] = jnp.maximum(normed, 0.0)


@jax.jit
def kernel(self_feat, neighbor_feats, W_self, b_self, W_nb, b_nb, ln_gamma, ln_beta):
    vec = pl.BlockSpec((D,), lambda i: (0,))
    mat = pl.BlockSpec((D, D), lambda i: (0, 0))
    return pl.pallas_call(
        _body,
        grid=(pl.cdiv(N, BN),),
        in_specs=[
            pl.BlockSpec((BN, D), lambda i: (i, 0)),
            pl.BlockSpec((K, BN, D), lambda i: (0, i, 0)),
            mat, vec, mat, vec, vec, vec,
        ],
        out_specs=pl.BlockSpec((BN, D), lambda i: (i, 0)),
        out_shape=jax.ShapeDtypeStruct((N, D), jnp.float32),
        compiler_params=pltpu.CompilerParams(
            dimension_semantics=("arbitrary",),
        ),
    )(self_feat, neighbor_feats, W_self, b_self, W_nb, b_nb, ln_gamma,
      ln_beta)


# speedup vs baseline: 1.0107x; 1.0011x over previous
import jax
import jax.numpy as jnp
from jax.experimental import pallas as pl
from jax.experimental.pallas import tpu as pltpu

N = 10000
K = 32
D = 128
BN = 440


def _body(self_ref, nf_ref, ws_ref, bs_ref, wn_ref, bn_ref, g_ref,
          beta_ref, out_ref):
    agg = jnp.sum(nf_ref[...], axis=0) * (1.0 / K)
    out = (
        jax.lax.dot_general(self_ref[...], ws_ref[...],
                            (((1,), (1,)), ((), ())),
                            preferred_element_type=jnp.float32)
        + jax.lax.dot_general(agg, wn_ref[...], (((1,), (1,)), ((), ())),
                              preferred_element_type=jnp.float32)
        + bs_ref[...] + bn_ref[...]
    )
    mu = jnp.mean(out, axis=-1, keepdims=True)
    var = jnp.mean(jnp.square(out - mu), axis=-1, keepdims=True)
    normed = (out - mu) * jax.lax.rsqrt(var + 1e-5) * g_ref[...] + beta_ref[...]
    out_ref[...] = jnp.maximum(normed, 0.0)


@jax.jit
def kernel(self_feat, neighbor_feats, W_self, b_self, W_nb, b_nb, ln_gamma, ln_beta):
    vec = pl.BlockSpec((D,), lambda i: (0,))
    mat = pl.BlockSpec((D, D), lambda i: (0, 0))
    return pl.pallas_call(
        _body,
        grid=(pl.cdiv(N, BN),),
        in_specs=[
            pl.BlockSpec((BN, D), lambda i: (i, 0)),
            pl.BlockSpec((K, BN, D), lambda i: (0, i, 0)),
            mat, vec, mat, vec, vec, vec,
        ],
        out_specs=pl.BlockSpec((BN, D), lambda i: (i, 0)),
        out_shape=jax.ShapeDtypeStruct((N, D), jnp.float32),
        compiler_params=pltpu.CompilerParams(
            dimension_semantics=("arbitrary",),
        ),
    )(self_feat, neighbor_feats, W_self, b_self, W_nb, b_nb, ln_gamma,
      ln_beta)
